# Initial kernel scaffold; baseline (speedup 1.0000x reference)
import jax
import jax.numpy as jnp
from jax.experimental import pallas as pl


def kernel(unique_node_ids_list, unique_messages_list, mem, last_update, W_C_w, W_C_b, W_h_w, W_h_b):
    raise NotImplementedError("write your pallas kernel here")


# speedup vs baseline: 2.4909x; 2.4909x over previous
import jax
import jax.numpy as jnp
from jax import lax
from jax.experimental import pallas as pl
from jax.experimental.pallas import tpu as pltpu
from jax.experimental.pallas import tpu_sc as plsc

N = 100000
D = 512
B = 50000
NC, NS = 2, 16
NW = NC * NS
CB = 1568
BPAD = NW * CB
NPAD = 100352
SH = NPAD // NW
NV = CB // 16
GCH = 112
SCH = 64

_MESH = plsc.VectorSubcoreMesh(
    core_axis_name="c", subcore_axis_name="s", num_cores=NC, num_subcores=NS
)
_SC_PARAMS = pltpu.CompilerParams(needs_layout_passes=False)


def _wid():
    return lax.axis_index("s") * NC + lax.axis_index("c")


def _gather_body(mem_ref, ids_hbm, rows_hbm, idsbuf, rowbuf, sem_g, sem_s):
    base = pl.multiple_of(_wid() * CB, CB)
    pltpu.sync_copy(ids_hbm.at[pl.ds(base, CB)], idsbuf)

    def trip(c, _):
        off = pl.multiple_of(c * GCH, GCH)
        pltpu.async_copy(mem_ref.at[idsbuf.at[pl.ds(off, GCH)]], rowbuf, sem_g).wait()
        pltpu.async_copy(rowbuf, rows_hbm.at[pl.ds(base + off, GCH)], sem_s).wait()
        return 0

    lax.fori_loop(0, CB // GCH, trip, 0)


_gather = pl.kernel(
    _gather_body,
    out_type=jax.ShapeDtypeStruct((BPAD, D), jnp.float32),
    mesh=_MESH,
    compiler_params=_SC_PARAMS,
    scratch_types=[
        pltpu.VMEM((CB,), jnp.int32),
        pltpu.VMEM((GCH, D), jnp.float32),
        pltpu.SemaphoreType.DMA,
        pltpu.SemaphoreType.DMA,
    ],
)


def _dedup_body(ids_hbm, tabs_hbm, idsbuf, tab):
    base = pl.multiple_of(_wid() * CB, CB)
    lanes = jnp.arange(16, dtype=jnp.int32)

    def initv(i, _):
        tab[pl.ds(i * 16, 16)] = jnp.full((16,), -1, jnp.int32)
        return 0

    lax.fori_loop(0, NPAD // 16, initv, 0)
    pltpu.sync_copy(ids_hbm.at[pl.ds(base, CB)], idsbuf)

    def step(i, _):
        ids16 = idsbuf[pl.ds(i * 16, 16)]
        pos = (base + i * 16) + lanes

        plsc.store_scatter(tab, [ids16], pos, mask=jnp.full((16,), True))
        got = plsc.load_gather(tab, [ids16])

        @pl.when(jnp.any(got < pos))
        def _slow():
            def fix(j, _):
                g2 = plsc.load_gather(tab, [ids16])
                plsc.store_scatter(tab, [ids16], pos, mask=g2 < pos)
                return 0

            lax.fori_loop(0, 15, fix, 0)

        return 0

    lax.fori_loop(0, NV, step, 0)
    pltpu.sync_copy(tab, tabs_hbm.at[pl.ds(pl.multiple_of(_wid() * NPAD, NPAD), NPAD)])


_dedup = pl.kernel(
    _dedup_body,
    out_type=jax.ShapeDtypeStruct((NW * NPAD,), jnp.int32),
    mesh=_MESH,
    compiler_params=_SC_PARAMS,
    scratch_types=[
        pltpu.VMEM((CB,), jnp.int32),
        pltpu.VMEM((NPAD,), jnp.int32),
    ],
)


def _merge_body(tabs_hbm, merged_hbm, buf2, acc, sem):
    off = pl.multiple_of(_wid() * SH, SH)
    for w0 in range(0, NW, 8):
        descs = [
            pltpu.async_copy(tabs_hbm.at[pl.ds(t * NPAD + off, SH)],
                             buf2.at[pl.ds(t * SH, SH)], sem)
            for t in range(w0, w0 + 8)
        ]
        for d in descs:
            d.wait()

    def step(v, _):
        m = buf2[pl.ds(v * 16, 16)]
        for t in range(1, NW):
            m = jnp.maximum(m, buf2[pl.ds(t * SH + v * 16, 16)])
        acc[pl.ds(v * 16, 16)] = m
        return 0

    lax.fori_loop(0, SH // 16, step, 0)
    pltpu.sync_copy(acc, merged_hbm.at[pl.ds(off, SH)])


_merge = pl.kernel(
    _merge_body,
    out_type=jax.ShapeDtypeStruct((NPAD,), jnp.int32),
    mesh=_MESH,
    compiler_params=_SC_PARAMS,
    scratch_types=[
        pltpu.VMEM((NW * SH,), jnp.int32),
        pltpu.VMEM((SH,), jnp.int32),
        pltpu.SemaphoreType.DMA,
    ],
)


def _scatter_body(mem_ref, ids_hbm, merged_hbm, newrows_hbm,
                  idsbuf, mvals, plist, ilist, pch, ich, rowbuf,
                  sem_e, sem_g, sem_s):
    base = pl.multiple_of(_wid() * CB, CB)
    lanes = jnp.arange(16, dtype=jnp.int32)
    pltpu.sync_copy(ids_hbm.at[pl.ds(base, CB)], idsbuf)

    for c in range(CB // GCH):
        sl = pl.ds(c * GCH, GCH)
        pltpu.async_copy(merged_hbm.at[idsbuf.at[sl]], mvals.at[sl], sem_e).wait()

    def comp(i, cnt):
        sl = pl.ds(i * 16, 16)
        ids16 = idsbuf[sl]
        pos = (base + i * 16) + lanes
        m = (mvals[sl] == pos) & (pos < B)
        plsc.store_compressed(plist.at[pl.ds(cnt, 16)], pos, mask=m)
        plsc.store_compressed(ilist.at[pl.ds(cnt, 16)], ids16, mask=m)
        return cnt + jnp.sum(m.astype(jnp.int32))

    cnt = lax.fori_loop(0, NV, comp, jnp.int32(0))

    def san(i, _):
        sl = pl.ds(i * 16, 16)
        g = i * 16 + lanes
        valid = g < cnt
        ilist[sl] = jnp.where(valid, ilist[sl], N + (g & 127))
        plist[sl] = jnp.where(valid, plist[sl], g)
        return 0

    lax.fori_loop(0, (CB + SCH) // 16, san, 0)

    nch = (cnt + (SCH - 1)) // SCH

    def trip(k, _):
        @pl.when(k < nch)
        def _do():
            st = k * SCH
            for j in range(SCH // 16):
                pch[pl.ds(j * 16, 16)] = plist[pl.ds(st + j * 16, 16)]
                ich[pl.ds(j * 16, 16)] = ilist[pl.ds(st + j * 16, 16)]
            pltpu.async_copy(newrows_hbm.at[pch], rowbuf, sem_g).wait()
            pltpu.async_copy(rowbuf, mem_ref.at[ich], sem_s).wait()

        return 0

    lax.fori_loop(0, CB // SCH + 1, trip, 0)


_scatter = pl.kernel(
    _scatter_body,
    out_type=(),
    mesh=_MESH,
    compiler_params=_SC_PARAMS,
    scratch_types=[
        pltpu.VMEM((CB,), jnp.int32),
        pltpu.VMEM((CB,), jnp.int32),
        pltpu.VMEM((CB + SCH,), jnp.int32),
        pltpu.VMEM((CB + SCH,), jnp.int32),
        pltpu.VMEM((SCH,), jnp.int32),
        pltpu.VMEM((SCH,), jnp.int32),
        pltpu.VMEM((SCH, D), jnp.float32),
        pltpu.SemaphoreType.DMA,
        pltpu.SemaphoreType.DMA,
        pltpu.SemaphoreType.DMA,
    ],
)


BM = 2000


def _tc_body(msg_ref, rows_ref, wc_ref, wh_ref, b_ref, out_ref):
    msg = msg_ref[...]
    rows = rows_ref[...]
    dn = (((1,), (1,)), ((), ()))
    i_c = lax.dot_general(msg, wc_ref[...], dn, preferred_element_type=jnp.float32)
    h = lax.dot_general(rows, wh_ref[...], dn, preferred_element_type=jnp.float32)
    s = i_c + h + b_ref[...]
    gate = jax.nn.sigmoid(s[:, :D])
    h_c = jnp.tanh(s[:, D:])
    out_ref[...] = (1.0 - gate) * h_c + gate * rows


def _tc_update(msg, rows_pad, wc, wh, bias2d):
    return pl.pallas_call(
        _tc_body,
        grid=(B // BM,),
        in_specs=[
            pl.BlockSpec((BM, D), lambda i: (i, 0)),
            pl.BlockSpec((BM, D), lambda i: (i, 0)),
            pl.BlockSpec((2 * D, D), lambda i: (0, 0)),
            pl.BlockSpec((2 * D, D), lambda i: (0, 0)),
            pl.BlockSpec((1, 2 * D), lambda i: (0, 0)),
        ],
        out_specs=pl.BlockSpec((BM, D), lambda i: (i, 0)),
        out_shape=jax.ShapeDtypeStruct((B, D), jnp.float32),
    )(msg, rows_pad, wc, wh, bias2d)


def kernel(unique_node_ids_list, unique_messages_list, mem, last_update,
           W_C_w, W_C_b, W_h_w, W_h_b):
    ids_l = unique_node_ids_list.astype(jnp.int32)
    bias2d = (W_C_b + W_h_b).reshape(1, 2 * D)
    membig = jnp.concatenate(
        [mem, jnp.zeros((NPAD - N, D), jnp.float32)], axis=0
    )
    ref = jax.new_ref(membig)
    padids = N + jnp.arange(BPAD - B, dtype=jnp.int32)
    divide = ids_l.shape[0]
    for k in range(divide):
        idsp = jnp.concatenate([ids_l[k], padids])
        rows_pad = _gather(ref, idsp)
        newrows = _tc_update(unique_messages_list[k], rows_pad, W_C_w, W_h_w, bias2d)
        tabs = _dedup(idsp)
        merged = _merge(tabs)
        _scatter(ref, idsp, merged, newrows)
    out = ref[...]
    return out[:N], last_update
